# baseline (speedup 1.0000x reference)
import functools

import jax
import jax.numpy as jnp
from jax import lax
from jax.experimental import pallas as pl
from jax.experimental.pallas import tpu as pltpu
from jax.experimental.pallas import tpu_sc as plsc

B = 16384
H = 64
AL = 128
ML = 100
SL = 128
OUT_COLS = 576

NW = 32
BPW = B // NW
ICH = 128
NCH = BPW // ICH

_EPS = 1e-5


def _sc_gather(upidx, mpidx, user2_t, music2_t):
    mesh = plsc.VectorSubcoreMesh(core_axis_name="c", subcore_axis_name="s",
                                  num_cores=2, num_subcores=16)
    out128 = jax.ShapeDtypeStruct((B, 2 * H), jnp.float32)

    @functools.partial(
        pl.kernel,
        mesh=mesh,
        out_type=(out128, out128),
        compiler_params=pltpu.CompilerParams(use_tc_tiling_on_sc=True),
        scratch_types=[
            pltpu.VMEM((NCH, ICH), jnp.int32),
            pltpu.VMEM((NCH, ICH), jnp.int32),
            pltpu.VMEM((2 * ICH, 2 * H), jnp.float32),
            pltpu.VMEM((2 * ICH, 2 * H), jnp.float32),
            pltpu.SemaphoreType.DMA,
            pltpu.SemaphoreType.DMA,
            pltpu.SemaphoreType.DMA,
            pltpu.SemaphoreType.DMA,
        ],
    )
    def body(uid_h, mid_h, ut_h, mt_h, uo_h, mo_h,
             idx0, idx1, buf0, buf1, g0, g1, w0, w1):
        wid = lax.axis_index("s") * 2 + lax.axis_index("c")
        base = wid * BPW
        pltpu.sync_copy(uid_h.at[wid], idx0)
        pltpu.sync_copy(mid_h.at[wid], idx1)
        bufs = (buf0, buf1)
        gsems = (g0, g1)
        wsems = (w0, w1)
        tasks = [(ut_h, idx0, uo_h, sb) for sb in range(2)] + \
                [(mt_h, idx1, mo_h, sb) for sb in range(2)]
        pend_g = [None, None]
        pend_w = [None, None]
        for k, (tab_h, idxr, out_h, sb) in enumerate(tasks):
            p = k % 2
            if pend_w[p] is not None:
                pend_w[p].wait()
            pend_g[p] = [
                pltpu.async_copy(tab_h.at[idxr.at[2 * sb + j]],
                                 bufs[p].at[pl.ds(j * ICH, ICH)], gsems[p])
                for j in range(2)
            ]
            q = 1 - p
            if k >= 1 and pend_g[q] is not None:
                for cp in pend_g[q]:
                    cp.wait()
                pend_g[q] = None
                ptab, pidx, pout, psb = tasks[k - 1]
                pend_w[q] = pltpu.async_copy(
                    bufs[q], pout.at[pl.ds(base + psb * 2 * ICH, 2 * ICH)],
                    wsems[q])
        p = (len(tasks) - 1) % 2
        for cp in pend_g[p]:
            cp.wait()
        ptab, pidx, pout, psb = tasks[-1]
        pend_w[p] = pltpu.async_copy(
            bufs[p], pout.at[pl.ds(base + psb * 2 * ICH, 2 * ICH)], wsems[p])
        pend_w[0].wait()
        pend_w[1].wait()

    r = lambda x: jnp.reshape(x, (NW, NCH, ICH))
    return body(r(upidx), r(mpidx), user2_t, music2_t)


_NB = 16
_BB = B // _NB


def _stats(art, mom, feat):
    def body(a_ref, m_ref, f_ref, sa, qa, sm, qm, sf, qf):
        @pl.when(pl.program_id(0) == 0)
        def _():
            for r in (sa, qa, sm, qm, sf, qf):
                r[...] = jnp.zeros_like(r)

        for x_ref, s_ref, q_ref in ((a_ref, sa, qa), (m_ref, sm, qm),
                                    (f_ref, sf, qf)):
            x = x_ref[...]
            s_ref[...] += jnp.sum(x, axis=0, keepdims=True)
            q_ref[...] += jnp.sum(x * x, axis=0, keepdims=True)

    stat_spec = lambda k: pl.BlockSpec((1, k), lambda i: (0, 0))
    return pl.pallas_call(
        body,
        grid=(_NB,),
        in_specs=[
            pl.BlockSpec((_BB, AL), lambda i: (i, 0)),
            pl.BlockSpec((_BB, ML), lambda i: (i, 0)),
            pl.BlockSpec((_BB, SL), lambda i: (i, 0)),
        ],
        out_specs=[stat_spec(AL), stat_spec(AL), stat_spec(ML),
                   stat_spec(ML), stat_spec(SL), stat_spec(SL)],
        out_shape=[jax.ShapeDtypeStruct((1, k), jnp.float32)
                   for k in (AL, AL, ML, ML, SL, SL)],
    )(art, mom, feat)


def _dense(upairs, mpairs, idx6, age_t, gender_t, singer_t, genre_t,
           art, mom, feat,
           w_uf, b_uf, w_ml, b_ml, w_sf, b_sf,
           g_art, be_art, g_mom, be_mom, g_feat, be_feat,
           sa, qa, sm, qm, sf, qf):
    def body(up_ref, mp_ref, ix_ref, at_ref, gt_ref, st_ref, grt_ref,
             a_ref, m_ref, f_ref,
             wa, ba, wm, bm, wf, bf,
             ga, bea, gm, bem, gf, bef,
             sa_r, qa_r, sm_r, qm_r, sf_r, qf_r, out_ref):
        def lookup(row, n, tab_ref):
            iota = lax.broadcasted_iota(jnp.int32, (n, _BB), 0)
            oh_t = jnp.where(ix_ref[row:row + 1, :] == iota.astype(jnp.float32),
                             1.0, 0.0)
            return jnp.einsum("kb,kh->bh", oh_t, tab_ref[...],
                              preferred_element_type=jnp.float32)

        def pick(p_ref, row):
            h = jnp.transpose(ix_ref[row:row + 1, :], (1, 0))
            lo = p_ref[:, 0:H]
            return lo + h * (p_ref[:, H:2 * H] - lo)

        out_ref[:, 0:H] = pick(up_ref, 4)
        out_ref[:, H:2 * H] = lookup(0, 6, at_ref)
        out_ref[:, 2 * H:3 * H] = lookup(1, 2, gt_ref)
        for x_ref, w_ref, b_ref, g_ref, be_ref, s_ref, q_ref, off in (
                (a_ref, wa, ba, ga, bea, sa_r, qa_r, 3 * H),
                (m_ref, wm, bm, gm, bem, sm_r, qm_r, 4 * H),
                (f_ref, wf, bf, gf, bef, sf_r, qf_r, 5 * H)):
            mu = s_ref[...] * (1.0 / B)
            var = q_ref[...] * (1.0 / B) - mu * mu
            sc = g_ref[...] / jnp.sqrt(var + _EPS)
            sh = be_ref[...] - mu * sc
            xn = x_ref[...] * sc + sh
            y = jnp.dot(xn, w_ref[...], preferred_element_type=jnp.float32)
            out_ref[:, off:off + H] = y + b_ref[...]
        out_ref[:, 6 * H:7 * H] = lookup(2, 417, st_ref)
        out_ref[:, 7 * H:8 * H] = lookup(3, 18, grt_ref)
        out_ref[:, 8 * H:9 * H] = pick(mp_ref, 5)

    full = lambda r, c: pl.BlockSpec((r, c), lambda i: (0, 0))
    pair_spec = pl.BlockSpec((_BB, 2 * H), lambda i: (i, 0))
    return pl.pallas_call(
        body,
        grid=(_NB,),
        in_specs=[
            pair_spec, pair_spec,
            pl.BlockSpec((6, _BB), lambda i: (0, i)),
            full(6, H), full(2, H), full(417, H), full(18, H),
            pl.BlockSpec((_BB, AL), lambda i: (i, 0)),
            pl.BlockSpec((_BB, ML), lambda i: (i, 0)),
            pl.BlockSpec((_BB, SL), lambda i: (i, 0)),
            full(AL, H), full(1, H), full(ML, H), full(1, H),
            full(SL, H), full(1, H),
            full(1, AL), full(1, AL), full(1, ML), full(1, ML),
            full(1, SL), full(1, SL),
            full(1, AL), full(1, AL), full(1, ML), full(1, ML),
            full(1, SL), full(1, SL),
        ],
        out_specs=pl.BlockSpec((_BB, OUT_COLS), lambda i: (i, 0)),
        out_shape=jax.ShapeDtypeStruct((B, OUT_COLS), jnp.float32),
        compiler_params=pltpu.CompilerParams(
            fuse_transposed_lhs_in_matmul=True),
    )(upairs, mpairs, idx6, age_t, gender_t, singer_t, genre_t,
      art, mom, feat,
      w_uf, b_uf.reshape(1, H), w_ml, b_ml.reshape(1, H),
      w_sf, b_sf.reshape(1, H),
      g_art.reshape(1, AL), be_art.reshape(1, AL),
      g_mom.reshape(1, ML), be_mom.reshape(1, ML),
      g_feat.reshape(1, SL), be_feat.reshape(1, SL),
      sa, qa, sm, qm, sf, qf)


def kernel(user_id, user_age, user_gender, user_articles, user_moments,
           music_id, music_singer, music_genre, music_features,
           UserEmb, AgeEmb, GenderEmb, SingerEmb, GenreEmb, MusicEmb,
           W_uf, b_uf, W_ml, b_ml, W_sf, b_sf,
           g_art, beta_art, g_mom, beta_mom, g_feat, beta_feat):
    uid = user_id.astype(jnp.int32)
    mid = music_id.astype(jnp.int32)
    user2 = UserEmb.reshape(UserEmb.shape[0] // 2, 2 * H)
    music2 = MusicEmb.reshape(MusicEmb.shape[0] // 2, 2 * H)
    upairs, mpairs = _sc_gather(uid >> 1, mid >> 1, user2, music2)
    sa, qa, sm, qm, sf, qf = _stats(user_articles, user_moments,
                                    music_features)
    idx6 = jnp.stack([user_age, user_gender, music_singer, music_genre,
                      uid & 1, mid & 1]).astype(jnp.float32)
    return _dense(upairs, mpairs, idx6, AgeEmb, GenderEmb, SingerEmb,
                  GenreEmb, user_articles, user_moments, music_features,
                  W_uf, b_uf, W_ml, b_ml, W_sf, b_sf,
                  g_art, beta_art, g_mom, beta_mom, g_feat, beta_feat,
                  sa, qa, sm, qm, sf, qf)


# speedup vs baseline: 1.0748x; 1.0748x over previous
import functools

import jax
import jax.numpy as jnp
from jax import lax
from jax.experimental import pallas as pl
from jax.experimental.pallas import tpu as pltpu
from jax.experimental.pallas import tpu_sc as plsc

B = 16384
H = 64
AL = 128
ML = 100
SL = 128
OUT_COLS = 576

NW = 32
BPW = B // NW
ICH = 128
NCH = BPW // ICH

_EPS = 1e-5


def _sc_gather(uid, mid, user_t, music_t):
    mesh = plsc.VectorSubcoreMesh(core_axis_name="c", subcore_axis_name="s",
                                  num_cores=2, num_subcores=16)

    @functools.partial(
        pl.kernel,
        mesh=mesh,
        out_type=jax.ShapeDtypeStruct((B, 2 * H), jnp.float32),
        compiler_params=pltpu.CompilerParams(use_tc_tiling_on_sc=False),
        scratch_types=[
            pltpu.VMEM((NCH, ICH), jnp.int32),
            pltpu.VMEM((NCH, ICH), jnp.int32),
            pltpu.VMEM((BPW, H), jnp.float32),
            pltpu.VMEM((BPW, H), jnp.float32),
            pltpu.SemaphoreType.DMA,
            pltpu.SemaphoreType.DMA,
            pltpu.SemaphoreType.DMA,
            pltpu.SemaphoreType.DMA,
        ],
    )
    def body(uid_h, mid_h, ut_h, mt_h, out_h,
             idx0, idx1, buf0, buf1, g0, g1, w0, w1):
        wid = lax.axis_index("s") * 2 + lax.axis_index("c")
        base = wid * BPW
        pltpu.sync_copy(uid_h.at[wid], idx0)
        gu = [pltpu.async_copy(ut_h.at[idx0.at[j]],
                               buf0.at[pl.ds(j * ICH, ICH)], g0)
              for j in range(NCH)]
        pltpu.sync_copy(mid_h.at[wid], idx1)
        gm = [pltpu.async_copy(mt_h.at[idx1.at[j]],
                               buf1.at[pl.ds(j * ICH, ICH)], g1)
              for j in range(NCH)]
        for cp in gu:
            cp.wait()
        wu = pltpu.async_copy(
            buf0, out_h.at[pl.ds(base, BPW), pl.ds(0, H)], w0)
        for cp in gm:
            cp.wait()
        wm = pltpu.async_copy(
            buf1, out_h.at[pl.ds(base, BPW), pl.ds(H, H)], w1)
        wu.wait()
        wm.wait()

    r = lambda x: jnp.reshape(x, (NW, NCH, ICH))
    return body(r(uid), r(mid), user_t, music_t)


_NB = 8
_BB = B // _NB


def _stats(art, mom, feat):
    def body(a_ref, m_ref, f_ref, sa, qa, sm, qm, sf, qf):
        @pl.when(pl.program_id(0) == 0)
        def _():
            for r in (sa, qa, sm, qm, sf, qf):
                r[...] = jnp.zeros_like(r)

        for x_ref, s_ref, q_ref in ((a_ref, sa, qa), (m_ref, sm, qm),
                                    (f_ref, sf, qf)):
            x = x_ref[...]
            s_ref[...] += jnp.sum(x, axis=0, keepdims=True)
            q_ref[...] += jnp.sum(x * x, axis=0, keepdims=True)

    stat_spec = lambda k: pl.BlockSpec((1, k), lambda i: (0, 0))
    return pl.pallas_call(
        body,
        grid=(_NB,),
        in_specs=[
            pl.BlockSpec((_BB, AL), lambda i: (i, 0)),
            pl.BlockSpec((_BB, ML), lambda i: (i, 0)),
            pl.BlockSpec((_BB, SL), lambda i: (i, 0)),
        ],
        out_specs=[stat_spec(AL), stat_spec(AL), stat_spec(ML),
                   stat_spec(ML), stat_spec(SL), stat_spec(SL)],
        out_shape=[jax.ShapeDtypeStruct((1, k), jnp.float32)
                   for k in (AL, AL, ML, ML, SL, SL)],
    )(art, mom, feat)


def _dense(comb, idx4, age_t, gender_t, singer_t, genre_t, art, mom, feat,
           w_uf, b_uf, w_ml, b_ml, w_sf, b_sf,
           g_art, be_art, g_mom, be_mom, g_feat, be_feat,
           sa, qa, sm, qm, sf, qf):
    def body(cb_ref, ix_ref, at_ref, gt_ref, st_ref, grt_ref,
             a_ref, m_ref, f_ref,
             wa, ba, wm, bm, wf, bf,
             ga, bea, gm, bem, gf, bef,
             sa_r, qa_r, sm_r, qm_r, sf_r, qf_r, out_ref):
        def lookup(row, n, tab_ref):
            iota = lax.broadcasted_iota(jnp.int32, (n, _BB), 0)
            oh_t = jnp.where(ix_ref[row:row + 1, :] == iota.astype(jnp.float32),
                             1.0, 0.0)
            return jnp.einsum("kb,kh->bh", oh_t, tab_ref[...],
                              preferred_element_type=jnp.float32)

        out_ref[:, 0:H] = cb_ref[:, 0:H]
        out_ref[:, H:2 * H] = lookup(0, 6, at_ref)
        out_ref[:, 2 * H:3 * H] = lookup(1, 2, gt_ref)
        for x_ref, w_ref, b_ref, g_ref, be_ref, s_ref, q_ref, off in (
                (a_ref, wa, ba, ga, bea, sa_r, qa_r, 3 * H),
                (m_ref, wm, bm, gm, bem, sm_r, qm_r, 4 * H),
                (f_ref, wf, bf, gf, bef, sf_r, qf_r, 5 * H)):
            mu = s_ref[...] * (1.0 / B)
            var = q_ref[...] * (1.0 / B) - mu * mu
            sc = g_ref[...] / jnp.sqrt(var + _EPS)
            sh = be_ref[...] - mu * sc
            xn = x_ref[...] * sc + sh
            y = jnp.dot(xn, w_ref[...], preferred_element_type=jnp.float32)
            out_ref[:, off:off + H] = y + b_ref[...]
        out_ref[:, 6 * H:7 * H] = lookup(2, 417, st_ref)
        out_ref[:, 7 * H:8 * H] = lookup(3, 18, grt_ref)
        out_ref[:, 8 * H:9 * H] = cb_ref[:, H:2 * H]

    full = lambda r, c: pl.BlockSpec((r, c), lambda i: (0, 0))
    return pl.pallas_call(
        body,
        grid=(_NB,),
        in_specs=[
            pl.BlockSpec((_BB, 2 * H), lambda i: (i, 0)),
            pl.BlockSpec((4, _BB), lambda i: (0, i)),
            full(6, H), full(2, H), full(417, H), full(18, H),
            pl.BlockSpec((_BB, AL), lambda i: (i, 0)),
            pl.BlockSpec((_BB, ML), lambda i: (i, 0)),
            pl.BlockSpec((_BB, SL), lambda i: (i, 0)),
            full(AL, H), full(1, H), full(ML, H), full(1, H),
            full(SL, H), full(1, H),
            full(1, AL), full(1, AL), full(1, ML), full(1, ML),
            full(1, SL), full(1, SL),
            full(1, AL), full(1, AL), full(1, ML), full(1, ML),
            full(1, SL), full(1, SL),
        ],
        out_specs=pl.BlockSpec((_BB, OUT_COLS), lambda i: (i, 0)),
        out_shape=jax.ShapeDtypeStruct((B, OUT_COLS), jnp.float32),
        compiler_params=pltpu.CompilerParams(
            fuse_transposed_lhs_in_matmul=True),
    )(comb, idx4, age_t, gender_t, singer_t, genre_t, art, mom, feat,
      w_uf, b_uf.reshape(1, H), w_ml, b_ml.reshape(1, H),
      w_sf, b_sf.reshape(1, H),
      g_art.reshape(1, AL), be_art.reshape(1, AL),
      g_mom.reshape(1, ML), be_mom.reshape(1, ML),
      g_feat.reshape(1, SL), be_feat.reshape(1, SL),
      sa, qa, sm, qm, sf, qf)


def kernel(user_id, user_age, user_gender, user_articles, user_moments,
           music_id, music_singer, music_genre, music_features,
           UserEmb, AgeEmb, GenderEmb, SingerEmb, GenreEmb, MusicEmb,
           W_uf, b_uf, W_ml, b_ml, W_sf, b_sf,
           g_art, beta_art, g_mom, beta_mom, g_feat, beta_feat):
    sa, qa, sm, qm, sf, qf = _stats(user_articles, user_moments,
                                    music_features)
    comb = _sc_gather(user_id.astype(jnp.int32), music_id.astype(jnp.int32),
                      UserEmb, MusicEmb)
    idx4 = jnp.stack([user_age, user_gender, music_singer,
                      music_genre]).astype(jnp.float32)
    return _dense(comb, idx4, AgeEmb, GenderEmb, SingerEmb, GenreEmb,
                  user_articles, user_moments, music_features,
                  W_uf, b_uf, W_ml, b_ml, W_sf, b_sf,
                  g_art, beta_art, g_mom, beta_mom, g_feat, beta_feat,
                  sa, qa, sm, qm, sf, qf)


# speedup vs baseline: 1.0775x; 1.0025x over previous
import functools

import jax
import jax.numpy as jnp
from jax import lax
from jax.experimental import pallas as pl
from jax.experimental.pallas import tpu as pltpu
from jax.experimental.pallas import tpu_sc as plsc

B = 16384
H = 64
AL = 128
ML = 100
SL = 128
OUT_COLS = 576

NW = 32
BPW = B // NW
ICH = 128
NCH = BPW // ICH

_EPS = 1e-5


def _sc_gather(uid, mid, user_t, music_t):
    mesh = plsc.VectorSubcoreMesh(core_axis_name="c", subcore_axis_name="s",
                                  num_cores=2, num_subcores=16)

    @functools.partial(
        pl.kernel,
        mesh=mesh,
        out_type=jax.ShapeDtypeStruct((B, 2 * H), jnp.float32),
        compiler_params=pltpu.CompilerParams(use_tc_tiling_on_sc=False),
        scratch_types=[
            pltpu.VMEM((NCH, ICH), jnp.int32),
            pltpu.VMEM((NCH, ICH), jnp.int32),
            pltpu.VMEM((BPW, H), jnp.float32),
            pltpu.VMEM((BPW, H), jnp.float32),
            pltpu.SemaphoreType.DMA,
            pltpu.SemaphoreType.DMA,
            pltpu.SemaphoreType.DMA,
            pltpu.SemaphoreType.DMA,
        ],
    )
    def body(uid_h, mid_h, ut_h, mt_h, out_h,
             idx0, idx1, buf0, buf1, g0, g1, w0, w1):
        wid = lax.axis_index("s") * 2 + lax.axis_index("c")
        base = wid * BPW
        pltpu.sync_copy(uid_h.at[wid], idx0)
        gu = [pltpu.async_copy(ut_h.at[idx0.at[j]],
                               buf0.at[pl.ds(j * ICH, ICH)], g0)
              for j in range(NCH)]
        pltpu.sync_copy(mid_h.at[wid], idx1)
        gm = [pltpu.async_copy(mt_h.at[idx1.at[j]],
                               buf1.at[pl.ds(j * ICH, ICH)], g1)
              for j in range(NCH)]
        for cp in gu:
            cp.wait()
        wu = pltpu.async_copy(
            buf0, out_h.at[pl.ds(base, BPW), pl.ds(0, H)], w0)
        for cp in gm:
            cp.wait()
        wm = pltpu.async_copy(
            buf1, out_h.at[pl.ds(base, BPW), pl.ds(H, H)], w1)
        wu.wait()
        wm.wait()

    r = lambda x: jnp.reshape(x, (NW, NCH, ICH))
    return body(r(uid), r(mid), user_t, music_t)


_NB = 4
_BB = B // _NB


def _stats(art, mom, feat):
    def body(a_ref, m_ref, f_ref, sa, qa, sm, qm, sf, qf):
        @pl.when(pl.program_id(0) == 0)
        def _():
            for r in (sa, qa, sm, qm, sf, qf):
                r[...] = jnp.zeros_like(r)

        for x_ref, s_ref, q_ref in ((a_ref, sa, qa), (m_ref, sm, qm),
                                    (f_ref, sf, qf)):
            x = x_ref[...]
            s_ref[...] += jnp.sum(x, axis=0, keepdims=True)
            q_ref[...] += jnp.sum(x * x, axis=0, keepdims=True)

    stat_spec = lambda k: pl.BlockSpec((1, k), lambda i: (0, 0))
    return pl.pallas_call(
        body,
        grid=(_NB,),
        in_specs=[
            pl.BlockSpec((_BB, AL), lambda i: (i, 0)),
            pl.BlockSpec((_BB, ML), lambda i: (i, 0)),
            pl.BlockSpec((_BB, SL), lambda i: (i, 0)),
        ],
        out_specs=[stat_spec(AL), stat_spec(AL), stat_spec(ML),
                   stat_spec(ML), stat_spec(SL), stat_spec(SL)],
        out_shape=[jax.ShapeDtypeStruct((1, k), jnp.float32)
                   for k in (AL, AL, ML, ML, SL, SL)],
    )(art, mom, feat)


def _dense(comb, idx4, age_t, gender_t, singer_t, genre_t, art, mom, feat,
           w_uf, b_uf, w_ml, b_ml, w_sf, b_sf,
           g_art, be_art, g_mom, be_mom, g_feat, be_feat,
           sa, qa, sm, qm, sf, qf):
    def body(cb_ref, ix_ref, at_ref, gt_ref, st_ref, grt_ref,
             a_ref, m_ref, f_ref,
             wa, ba, wm, bm, wf, bf,
             ga, bea, gm, bem, gf, bef,
             sa_r, qa_r, sm_r, qm_r, sf_r, qf_r, out_ref):
        def lookup(row, n, tab_ref):
            iota = lax.broadcasted_iota(jnp.int32, (n, _BB), 0)
            oh_t = jnp.where(ix_ref[row:row + 1, :] == iota.astype(jnp.float32),
                             1.0, 0.0)
            return jnp.einsum("kb,kh->bh", oh_t, tab_ref[...],
                              preferred_element_type=jnp.float32)

        out_ref[:, 0:H] = cb_ref[:, 0:H]
        out_ref[:, H:2 * H] = lookup(0, 6, at_ref)
        out_ref[:, 2 * H:3 * H] = lookup(1, 2, gt_ref)
        for x_ref, w_ref, b_ref, g_ref, be_ref, s_ref, q_ref, off in (
                (a_ref, wa, ba, ga, bea, sa_r, qa_r, 3 * H),
                (m_ref, wm, bm, gm, bem, sm_r, qm_r, 4 * H),
                (f_ref, wf, bf, gf, bef, sf_r, qf_r, 5 * H)):
            mu = s_ref[...] * (1.0 / B)
            var = q_ref[...] * (1.0 / B) - mu * mu
            sc = g_ref[...] / jnp.sqrt(var + _EPS)
            sh = be_ref[...] - mu * sc
            xn = x_ref[...] * sc + sh
            y = jnp.dot(xn, w_ref[...], preferred_element_type=jnp.float32)
            out_ref[:, off:off + H] = y + b_ref[...]
        out_ref[:, 6 * H:7 * H] = lookup(2, 417, st_ref)
        out_ref[:, 7 * H:8 * H] = lookup(3, 18, grt_ref)
        out_ref[:, 8 * H:9 * H] = cb_ref[:, H:2 * H]

    full = lambda r, c: pl.BlockSpec((r, c), lambda i: (0, 0))
    return pl.pallas_call(
        body,
        grid=(_NB,),
        in_specs=[
            pl.BlockSpec((_BB, 2 * H), lambda i: (i, 0)),
            pl.BlockSpec((4, _BB), lambda i: (0, i)),
            full(6, H), full(2, H), full(417, H), full(18, H),
            pl.BlockSpec((_BB, AL), lambda i: (i, 0)),
            pl.BlockSpec((_BB, ML), lambda i: (i, 0)),
            pl.BlockSpec((_BB, SL), lambda i: (i, 0)),
            full(AL, H), full(1, H), full(ML, H), full(1, H),
            full(SL, H), full(1, H),
            full(1, AL), full(1, AL), full(1, ML), full(1, ML),
            full(1, SL), full(1, SL),
            full(1, AL), full(1, AL), full(1, ML), full(1, ML),
            full(1, SL), full(1, SL),
        ],
        out_specs=pl.BlockSpec((_BB, OUT_COLS), lambda i: (i, 0)),
        out_shape=jax.ShapeDtypeStruct((B, OUT_COLS), jnp.float32),
        compiler_params=pltpu.CompilerParams(
            fuse_transposed_lhs_in_matmul=True),
    )(comb, idx4, age_t, gender_t, singer_t, genre_t, art, mom, feat,
      w_uf, b_uf.reshape(1, H), w_ml, b_ml.reshape(1, H),
      w_sf, b_sf.reshape(1, H),
      g_art.reshape(1, AL), be_art.reshape(1, AL),
      g_mom.reshape(1, ML), be_mom.reshape(1, ML),
      g_feat.reshape(1, SL), be_feat.reshape(1, SL),
      sa, qa, sm, qm, sf, qf)


def kernel(user_id, user_age, user_gender, user_articles, user_moments,
           music_id, music_singer, music_genre, music_features,
           UserEmb, AgeEmb, GenderEmb, SingerEmb, GenreEmb, MusicEmb,
           W_uf, b_uf, W_ml, b_ml, W_sf, b_sf,
           g_art, beta_art, g_mom, beta_mom, g_feat, beta_feat):
    sa, qa, sm, qm, sf, qf = _stats(user_articles, user_moments,
                                    music_features)
    comb = _sc_gather(user_id.astype(jnp.int32), music_id.astype(jnp.int32),
                      UserEmb, MusicEmb)
    idx4 = jnp.stack([user_age, user_gender, music_singer,
                      music_genre]).astype(jnp.float32)
    return _dense(comb, idx4, AgeEmb, GenderEmb, SingerEmb, GenreEmb,
                  user_articles, user_moments, music_features,
                  W_uf, b_uf, W_ml, b_ml, W_sf, b_sf,
                  g_art, beta_art, g_mom, beta_mom, g_feat, beta_feat,
                  sa, qa, sm, qm, sf, qf)
